# baseline (speedup 1.0000x reference)
import functools

import jax
import jax.numpy as jnp
from jax import lax
from jax.experimental import pallas as pl
from jax.experimental.pallas import tpu as pltpu
from jax.experimental.pallas import tpu_sc as plsc

N_NODES = 10000
N_PAD = 10240
D_FEAT = 128
HIDDEN = 16
OUT_DIM = 1

NC = 2
NS = 16
NW = NC * NS
CHUNK = 128
E_PAD = 327680
NCHUNK = E_PAD // (NW * CHUNK)
ROWS_PER_TILE = N_PAD // NS


def _mesh():
    return plsc.VectorSubcoreMesh(core_axis_name="c", subcore_axis_name="s")


def _mm1_body(x_ref, w_ref, o_ref):
    o_ref[...] = jnp.dot(x_ref[...], w_ref[...],
                         preferred_element_type=jnp.float32)


def _mid_body(p_ref, b1_ref, w2_ref, o_ref):
    m1 = p_ref[0, :N_NODES, :] + p_ref[1, :N_NODES, :]
    h = jnp.maximum(m1 + b1_ref[...], 0.0)
    o_ref[...] = jnp.dot(h, w2_ref[...],
                         preferred_element_type=jnp.float32)


def _final_body(p_ref, b2_ref, o_ref):
    z = p_ref[0, :N_NODES] + p_ref[1, :N_NODES] + b2_ref[...]
    o_ref[...] = 1.0 / (1.0 + jnp.exp(-z))


def _spmm16_body(t1_hbm, col_hbm, row_hbm, w_hbm, out_hbm,
                 colv, rowv, wv, rows_v, buf_v, acc_sh, sem):
    c = lax.axis_index("c")
    s = lax.axis_index("s")
    wid = s * NC + c

    pltpu.sync_copy(col_hbm.at[wid], colv)
    pltpu.sync_copy(row_hbm.at[wid], rowv)
    pltpu.sync_copy(w_hbm.at[wid], wv)

    def _zero(i, _):
        buf_v[i] = jnp.zeros((16,), jnp.float32)
        return 0
    lax.fori_loop(0, ROWS_PER_TILE, _zero, 0)
    pltpu.sync_copy(buf_v, acc_sh.at[pl.ds(s * ROWS_PER_TILE, ROWS_PER_TILE)])
    plsc.subcore_barrier()

    def _chunk(j, _):
        pltpu.async_copy(t1_hbm.at[colv.at[j]], rows_v, sem).wait()

        def _scale(g, _):
            w16 = wv[j, pl.ds(g * 16, 16)]
            base = g * 16
            for k in range(16):
                rows_v[base + k] = rows_v[base + k] * w16[k]
            return 0
        lax.fori_loop(0, CHUNK // 16, _scale, 0)

        pltpu.sync_copy(rows_v, acc_sh.at[rowv.at[j]], add=True)
        return 0

    lax.fori_loop(0, NCHUNK, _chunk, 0)
    plsc.subcore_barrier()

    pltpu.sync_copy(acc_sh.at[pl.ds(s * ROWS_PER_TILE, ROWS_PER_TILE)], buf_v)
    pltpu.sync_copy(buf_v, out_hbm.at[c, pl.ds(s * ROWS_PER_TILE, ROWS_PER_TILE)])


def _spmm1_body(t2_hbm, col_hbm, row_hbm, w_hbm, out_hbm,
                t2v, colv, rowv, wv, vals_v, buf_v, acc_sh, sem):
    c = lax.axis_index("c")
    s = lax.axis_index("s")
    wid = s * NC + c

    pltpu.sync_copy(t2_hbm, t2v)
    pltpu.sync_copy(col_hbm.at[wid], colv)
    pltpu.sync_copy(row_hbm.at[wid], rowv)
    pltpu.sync_copy(w_hbm.at[wid], wv)

    def _zero(i, _):
        buf_v[pl.ds(i * 16, 16)] = jnp.zeros((16,), jnp.float32)
        return 0
    lax.fori_loop(0, ROWS_PER_TILE // 16, _zero, 0)
    pltpu.sync_copy(buf_v, acc_sh.at[pl.ds(s * ROWS_PER_TILE, ROWS_PER_TILE)])
    plsc.subcore_barrier()

    def _chunk(j, _):
        for g in range(CHUNK // 16):
            col16 = colv[j, pl.ds(g * 16, 16)]
            w16 = wv[j, pl.ds(g * 16, 16)]
            vals = plsc.load_gather(t2v, [col16])
            vals_v[pl.ds(g * 16, 16)] = vals * w16
        pltpu.sync_copy(vals_v, acc_sh.at[rowv.at[j]], add=True)
        return 0

    lax.fori_loop(0, NCHUNK, _chunk, 0)
    plsc.subcore_barrier()

    pltpu.sync_copy(acc_sh.at[pl.ds(s * ROWS_PER_TILE, ROWS_PER_TILE)], buf_v)
    pltpu.sync_copy(buf_v, out_hbm.at[c, pl.ds(s * ROWS_PER_TILE, ROWS_PER_TILE)])


@functools.partial(
    pl.kernel,
    out_type=jax.ShapeDtypeStruct((NC, N_PAD, HIDDEN), jnp.float32),
    mesh=_mesh(),
    compiler_params=pltpu.CompilerParams(use_tc_tiling_on_sc=False, needs_layout_passes=False),
    scratch_types=[
        pltpu.VMEM((NCHUNK, CHUNK), jnp.int32),
        pltpu.VMEM((NCHUNK, CHUNK), jnp.int32),
        pltpu.VMEM((NCHUNK, CHUNK), jnp.float32),
        pltpu.VMEM((CHUNK, HIDDEN), jnp.float32),
        pltpu.VMEM((ROWS_PER_TILE, HIDDEN), jnp.float32),
        pltpu.VMEM_SHARED((N_PAD, HIDDEN), jnp.float32),
        pltpu.SemaphoreType.DMA,
    ],
)
def _spmm16(t1_hbm, col_hbm, row_hbm, w_hbm, out_hbm,
            colv, rowv, wv, rows_v, buf_v, acc_sh, sem):
    _spmm16_body(t1_hbm, col_hbm, row_hbm, w_hbm, out_hbm,
                 colv, rowv, wv, rows_v, buf_v, acc_sh, sem)


@functools.partial(
    pl.kernel,
    out_type=jax.ShapeDtypeStruct((NC, N_PAD), jnp.float32),
    mesh=_mesh(),
    compiler_params=pltpu.CompilerParams(use_tc_tiling_on_sc=False, needs_layout_passes=False),
    scratch_types=[
        pltpu.VMEM((N_NODES,), jnp.float32),
        pltpu.VMEM((NCHUNK, CHUNK), jnp.int32),
        pltpu.VMEM((NCHUNK, CHUNK), jnp.int32),
        pltpu.VMEM((NCHUNK, CHUNK), jnp.float32),
        pltpu.VMEM((CHUNK,), jnp.float32),
        pltpu.VMEM((ROWS_PER_TILE,), jnp.float32),
        pltpu.VMEM_SHARED((N_PAD,), jnp.float32),
        pltpu.SemaphoreType.DMA,
    ],
)
def _spmm1(t2_hbm, col_hbm, row_hbm, w_hbm, out_hbm,
           t2v, colv, rowv, wv, vals_v, buf_v, acc_sh, sem):
    _spmm1_body(t2_hbm, col_hbm, row_hbm, w_hbm, out_hbm,
                t2v, colv, rowv, wv, vals_v, buf_v, acc_sh, sem)


def kernel(x, edge_index, edge_weight, W1, b1, W2, b2):
    row = edge_index[0]
    col = edge_index[1]
    n_edges = row.shape[0]
    pad = E_PAD - n_edges
    pad_idx = (jnp.arange(pad, dtype=jnp.int32) * 7) % N_NODES
    colp = jnp.concatenate([col, pad_idx]).reshape(NW, NCHUNK, CHUNK)
    rowp = jnp.concatenate([row, pad_idx]).reshape(NW, NCHUNK, CHUNK)
    wp = jnp.concatenate(
        [edge_weight, jnp.zeros((pad,), jnp.float32)]).reshape(NW, NCHUNK, CHUNK)

    t1 = pl.pallas_call(
        _mm1_body,
        out_shape=jax.ShapeDtypeStruct((N_NODES, HIDDEN), jnp.float32),
    )(x, W1)

    parts1 = _spmm16(t1, colp, rowp, wp)

    t2 = pl.pallas_call(
        _mid_body,
        out_shape=jax.ShapeDtypeStruct((N_NODES, OUT_DIM), jnp.float32),
    )(parts1, b1, W2).reshape(N_NODES)

    parts2 = _spmm1(t2, colp, rowp, wp)

    out = pl.pallas_call(
        _final_body,
        out_shape=jax.ShapeDtypeStruct((N_NODES,), jnp.float32),
    )(parts2, b2)

    return out.reshape(N_NODES, OUT_DIM)


# speedup vs baseline: 1.3926x; 1.3926x over previous
import functools

import jax
import jax.numpy as jnp
from jax import lax
from jax.experimental import pallas as pl
from jax.experimental.pallas import tpu as pltpu
from jax.experimental.pallas import tpu_sc as plsc

N_NODES = 10000
N_PAD = 10240
D_FEAT = 128
HIDDEN = 16
OUT_DIM = 1

NC = 2
NS = 16
NW = NC * NS
CHUNK = 128
E_PAD = 327680
NCHUNK = E_PAD // (NW * CHUNK)
ROWS_PER_TILE = N_PAD // NS
NBUF = 4


def _mesh():
    return plsc.VectorSubcoreMesh(core_axis_name="c", subcore_axis_name="s")


def _mm1_body(x_ref, w_ref, o_ref):
    o_ref[...] = jnp.dot(x_ref[...], w_ref[...],
                         preferred_element_type=jnp.float32)


def _mid_body(p_ref, b1_ref, w2_ref, o_ref):
    m1 = p_ref[0, :N_NODES, :] + p_ref[1, :N_NODES, :]
    h = jnp.maximum(m1 + b1_ref[...], 0.0)
    o_ref[...] = jnp.dot(h, w2_ref[...],
                         preferred_element_type=jnp.float32)


def _final_body(p_ref, b2_ref, o_ref):
    z = p_ref[0, :N_NODES] + p_ref[1, :N_NODES] + b2_ref[...]
    o_ref[...] = 1.0 / (1.0 + jnp.exp(-z))


def _spmm16_body(t1_hbm, col_hbm, row_hbm, w_hbm, out_hbm,
                 colv, rowv, wv, rows_v, buf_v, acc_sh,
                 g0, g1, g2, g3, s0, s1, s2, s3):
    gsems = (g0, g1, g2, g3)
    ssems = (s0, s1, s2, s3)
    c = lax.axis_index("c")
    s = lax.axis_index("s")
    wid = s * NC + c

    pltpu.sync_copy(col_hbm.at[wid], colv)
    pltpu.sync_copy(row_hbm.at[wid], rowv)
    pltpu.sync_copy(w_hbm.at[wid], wv)

    def _zero(i, _):
        buf_v[i] = jnp.zeros((16,), jnp.float32)
        return 0
    lax.fori_loop(0, ROWS_PER_TILE, _zero, 0)
    pltpu.sync_copy(buf_v, acc_sh.at[pl.ds(s * ROWS_PER_TILE, ROWS_PER_TILE)])
    plsc.subcore_barrier()

    def _gstart(j, b):
        pltpu.async_copy(t1_hbm.at[colv.at[j]], rows_v.at[b], gsems[b])

    def _gwait(j, b):
        pltpu.make_async_copy(t1_hbm.at[colv.at[j]], rows_v.at[b],
                              gsems[b]).wait()

    def _sstart(j, b):
        pltpu.async_copy(rows_v.at[b], acc_sh.at[rowv.at[j]], ssems[b],
                         add=True)

    def _swait(j, b):
        pltpu.make_async_copy(rows_v.at[b], acc_sh.at[rowv.at[j]],
                              ssems[b]).wait()

    _gstart(0, 0)
    _gstart(1, 1)

    def _outer(jj, _):
        for b in range(NBUF):
            j = jj * NBUF + b
            _gwait(j, b)

            def _scale(g, _):
                w16 = wv[j, pl.ds(g * 16, 16)]
                base = g * 16
                for k in range(16):
                    rows_v[b, base + k] = rows_v[b, base + k] * w16[k]
                return 0
            lax.fori_loop(0, CHUNK // 16, _scale, 0)

            _sstart(j, b)

            bt = (b + 2) % NBUF

            @pl.when(j >= 2)
            def _():
                _swait(j - 2, bt)

            @pl.when(j + 2 < NCHUNK)
            def _():
                _gstart(j + 2, bt)
        return 0

    lax.fori_loop(0, NCHUNK // NBUF, _outer, 0)
    _swait(NCHUNK - 2, (NCHUNK - 2) % NBUF)
    _swait(NCHUNK - 1, (NCHUNK - 1) % NBUF)
    plsc.subcore_barrier()

    pltpu.sync_copy(acc_sh.at[pl.ds(s * ROWS_PER_TILE, ROWS_PER_TILE)], buf_v)
    pltpu.sync_copy(buf_v, out_hbm.at[c, pl.ds(s * ROWS_PER_TILE, ROWS_PER_TILE)])


def _spmm1_body(t2_hbm, col_hbm, row_hbm, w_hbm, out_hbm,
                t2v, colv, rowv, wv, vals_v, buf_v, acc_sh, *sems):
    c = lax.axis_index("c")
    s = lax.axis_index("s")
    wid = s * NC + c

    pltpu.sync_copy(t2_hbm, t2v)
    pltpu.sync_copy(col_hbm.at[wid], colv)
    pltpu.sync_copy(row_hbm.at[wid], rowv)
    pltpu.sync_copy(w_hbm.at[wid], wv)

    def _zero(i, _):
        buf_v[pl.ds(i * 16, 16)] = jnp.zeros((16,), jnp.float32)
        return 0
    lax.fori_loop(0, ROWS_PER_TILE // 16, _zero, 0)
    pltpu.sync_copy(buf_v, acc_sh.at[pl.ds(s * ROWS_PER_TILE, ROWS_PER_TILE)])
    plsc.subcore_barrier()

    def _sstart1(j, b):
        pltpu.async_copy(vals_v.at[b], acc_sh.at[rowv.at[j]], sems[b],
                         add=True)

    def _swait1(j, b):
        pltpu.make_async_copy(vals_v.at[b], acc_sh.at[rowv.at[j]],
                              sems[b]).wait()

    def _outer(jj, _):
        for b in range(2):
            j = jj * 2 + b
            for g in range(CHUNK // 16):
                col16 = colv[j, pl.ds(g * 16, 16)]
                w16 = wv[j, pl.ds(g * 16, 16)]
                vals = plsc.load_gather(t2v, [col16])
                vals_v[b, pl.ds(g * 16, 16)] = vals * w16

            @pl.when(j >= 2)
            def _():
                _swait1(j - 2, b)

            _sstart1(j, b)
        return 0

    lax.fori_loop(0, NCHUNK // 2, _outer, 0)
    _swait1(NCHUNK - 2, 0)
    _swait1(NCHUNK - 1, 1)
    plsc.subcore_barrier()

    pltpu.sync_copy(acc_sh.at[pl.ds(s * ROWS_PER_TILE, ROWS_PER_TILE)], buf_v)
    pltpu.sync_copy(buf_v, out_hbm.at[c, pl.ds(s * ROWS_PER_TILE, ROWS_PER_TILE)])


@functools.partial(
    pl.kernel,
    out_type=jax.ShapeDtypeStruct((NC, N_PAD, HIDDEN), jnp.float32),
    mesh=_mesh(),
    compiler_params=pltpu.CompilerParams(use_tc_tiling_on_sc=False, needs_layout_passes=False),
    scratch_types=[
        pltpu.VMEM((NCHUNK, CHUNK), jnp.int32),
        pltpu.VMEM((NCHUNK, CHUNK), jnp.int32),
        pltpu.VMEM((NCHUNK, CHUNK), jnp.float32),
        pltpu.VMEM((NBUF, CHUNK, HIDDEN), jnp.float32),
        pltpu.VMEM((ROWS_PER_TILE, HIDDEN), jnp.float32),
        pltpu.VMEM_SHARED((N_PAD, HIDDEN), jnp.float32),
    ] + [pltpu.SemaphoreType.DMA] * (2 * NBUF),
)
def _spmm16(t1_hbm, col_hbm, row_hbm, w_hbm, out_hbm,
            colv, rowv, wv, rows_v, buf_v, acc_sh, *sems):
    _spmm16_body(t1_hbm, col_hbm, row_hbm, w_hbm, out_hbm,
                 colv, rowv, wv, rows_v, buf_v, acc_sh, *sems)


@functools.partial(
    pl.kernel,
    out_type=jax.ShapeDtypeStruct((NC, N_PAD), jnp.float32),
    mesh=_mesh(),
    compiler_params=pltpu.CompilerParams(use_tc_tiling_on_sc=False, needs_layout_passes=False),
    scratch_types=[
        pltpu.VMEM((N_NODES,), jnp.float32),
        pltpu.VMEM((NCHUNK, CHUNK), jnp.int32),
        pltpu.VMEM((NCHUNK, CHUNK), jnp.int32),
        pltpu.VMEM((NCHUNK, CHUNK), jnp.float32),
        pltpu.VMEM((2, CHUNK), jnp.float32),
        pltpu.VMEM((ROWS_PER_TILE,), jnp.float32),
        pltpu.VMEM_SHARED((N_PAD,), jnp.float32),
        pltpu.SemaphoreType.DMA,
        pltpu.SemaphoreType.DMA,
    ],
)
def _spmm1(t2_hbm, col_hbm, row_hbm, w_hbm, out_hbm,
           t2v, colv, rowv, wv, vals_v, buf_v, acc_sh, *sems):
    _spmm1_body(t2_hbm, col_hbm, row_hbm, w_hbm, out_hbm,
                t2v, colv, rowv, wv, vals_v, buf_v, acc_sh, *sems)


def kernel(x, edge_index, edge_weight, W1, b1, W2, b2):
    row = edge_index[0]
    col = edge_index[1]
    n_edges = row.shape[0]
    pad = E_PAD - n_edges
    pad_idx = (jnp.arange(pad, dtype=jnp.int32) * 7) % N_NODES
    colp = jnp.concatenate([col, pad_idx]).reshape(NW, NCHUNK, CHUNK)
    rowp = jnp.concatenate([row, pad_idx]).reshape(NW, NCHUNK, CHUNK)
    wp = jnp.concatenate(
        [edge_weight, jnp.zeros((pad,), jnp.float32)]).reshape(NW, NCHUNK, CHUNK)

    t1 = pl.pallas_call(
        _mm1_body,
        out_shape=jax.ShapeDtypeStruct((N_NODES, HIDDEN), jnp.float32),
    )(x, W1)

    parts1 = _spmm16(t1, colp, rowp, wp)

    t2 = pl.pallas_call(
        _mid_body,
        out_shape=jax.ShapeDtypeStruct((N_NODES, OUT_DIM), jnp.float32),
    )(parts1, b1, W2).reshape(N_NODES)

    parts2 = _spmm1(t2, colp, rowp, wp)

    out = pl.pallas_call(
        _final_body,
        out_shape=jax.ShapeDtypeStruct((N_NODES,), jnp.float32),
    )(parts2, b2)

    return out.reshape(N_NODES, OUT_DIM)
